# baseline (speedup 1.0000x reference)
import functools

import jax
import jax.numpy as jnp
from jax import lax
from jax.experimental import pallas as pl
from jax.experimental.pallas import tpu as pltpu
from jax.experimental.pallas import tpu_sc as plsc

_N = 10000
_E = 320000
_D = 128
_DE = 16

_NODE_BLK = 1000
_EDGE_BLK = 4000

_NC = 2
_NS = 16
_NW = _NC * _NS
_EPW = _E // _NW
_C = 40
_NCHUNK = _EPW // _C
_NPAD = 10240
_RPS = _NPAD // _NS


def _node_mm_body(x_ref, w_ref, b_ref, o_ref):
    acc = jnp.dot(x_ref[...], w_ref[...], preferred_element_type=jnp.float32)
    o_ref[...] = acc + b_ref[...]


def _node_matmuls(x, w_cat, b_cat):
    return pl.pallas_call(
        _node_mm_body,
        grid=(_N // _NODE_BLK,),
        in_specs=[
            pl.BlockSpec((_NODE_BLK, _D), lambda i: (i, 0)),
            pl.BlockSpec((_D, 4 * _D), lambda i: (0, 0)),
            pl.BlockSpec((1, 4 * _D), lambda i: (0, 0)),
        ],
        out_specs=pl.BlockSpec((_NODE_BLK, 4 * _D), lambda i: (i, 0)),
        out_shape=jax.ShapeDtypeStruct((_N, 4 * _D), jnp.float32),
    )(x, w_cat, b_cat)


def _edge_mm_body(a_ref, w_ref, b_ref, o_ref):
    acc = jnp.dot(a_ref[...], w_ref[...], preferred_element_type=jnp.float32)
    o_ref[...] = acc + b_ref[...]


def _edge_matmul(edge_attr, we_t, be_row):
    return pl.pallas_call(
        _edge_mm_body,
        grid=(_E // _EDGE_BLK,),
        in_specs=[
            pl.BlockSpec((_EDGE_BLK, _DE), lambda i: (i, 0)),
            pl.BlockSpec((_DE, _D), lambda i: (0, 0)),
            pl.BlockSpec((1, _D), lambda i: (0, 0)),
        ],
        out_specs=pl.BlockSpec((_EDGE_BLK, _D), lambda i: (i, 0)),
        out_shape=jax.ShapeDtypeStruct((_E, _D), jnp.float32),
    )(edge_attr, we_t, be_row)


def _sc_edge_body(k_hbm, q_hbm, v_hbm, e_hbm, src_hbm, dst_hbm,
                  out0_hbm, out1_hbm,
                  srcv0, srcv1, dstv0, dstv1,
                  kb0, kb1, qb0, qb1, vb0, vb1, eb0, eb1,
                  aggsh, semG0, semG1, semE0, semE1, semS0, semS1):
    cid = lax.axis_index("c")
    sid = lax.axis_index("s")
    wid = sid * _NC + cid
    ebase = wid * _EPW

    srcv = (srcv0, srcv1)
    dstv = (dstv0, dstv1)
    kb = (kb0, kb1)
    qb = (qb0, qb1)
    vb = (vb0, vb1)
    eb = (eb0, eb1)
    semG = (semG0, semG1)
    semE = (semE0, semE1)
    semS = (semS0, semS1)

    zero = jnp.zeros((16,), jnp.float32)

    def zrow(i, _):
        for r in range(_D // 16):
            eb0[i, pl.ds(r * 16, 16)] = zero
        return 0

    lax.fori_loop(0, _C, zrow, 0)
    zbase = pl.multiple_of(sid * _RPS, 8)
    for m in range(_RPS // _C):
        pltpu.sync_copy(eb0, aggsh.at[pl.ds(zbase + m * _C, _C)])
    plsc.subcore_barrier()

    def chunk_off(t):
        return pl.multiple_of(ebase + t * _C, 8)

    def issue_pre(t, b):
        off = chunk_off(t)
        pltpu.async_copy(e_hbm.at[pl.ds(off, _C)], eb[b], semE[b])
        pltpu.async_copy(src_hbm.at[pl.ds(off, _C)], srcv[b], semE[b])
        pltpu.async_copy(dst_hbm.at[pl.ds(off, _C)], dstv[b], semE[b])

    def wait_pre(b):
        pltpu.make_async_copy(e_hbm.at[pl.ds(0, _C)], eb[b], semE[b]).wait()
        pltpu.make_async_copy(src_hbm.at[pl.ds(0, _C)], srcv[b], semE[b]).wait()
        pltpu.make_async_copy(dst_hbm.at[pl.ds(0, _C)], dstv[b], semE[b]).wait()

    def issue_gathers(b):
        pltpu.async_copy(k_hbm.at[dstv[b]], kb[b], semG[b])
        pltpu.async_copy(q_hbm.at[srcv[b]], qb[b], semG[b])
        pltpu.async_copy(v_hbm.at[srcv[b]], vb[b], semG[b])

    def wait_gathers(b):
        pltpu.make_async_copy(k_hbm.at[pl.ds(0, _C)], kb[b], semG[b]).wait()
        pltpu.make_async_copy(q_hbm.at[pl.ds(0, _C)], qb[b], semG[b]).wait()
        pltpu.make_async_copy(v_hbm.at[pl.ds(0, _C)], vb[b], semG[b]).wait()

    def compute(b):
        kp, qp, vp, ep = kb[b], qb[b], vb[b], eb[b]

        @plsc.parallel_loop(0, _C, 1, unroll=2)
        def _(i):
            for r in range(_D // 16):
                sl = pl.ds(r * 16, 16)
                z = kp[i, sl] + ep[i, sl] + qp[i, sl]
                vp[i, sl] = vp[i, sl] / (1.0 + jnp.exp(-z))

    def issue_scatter(b):
        pltpu.async_copy(vb[b], aggsh.at[dstv[b]], semS[b], add=True)

    def wait_scatter(b):
        pltpu.make_async_copy(e_hbm.at[pl.ds(0, _C)], vb[b], semS[b]).wait()

    def steady(t, p, last=False):
        q = 1 - p
        wait_scatter(q)
        if not last:
            issue_pre(t + 1, q)
        wait_gathers(p)
        compute(p)
        if not last:
            wait_pre(q)
            issue_gathers(q)
        issue_scatter(p)

    pltpu.sync_copy(src_hbm.at[pl.ds(chunk_off(0), _C)], srcv0)
    pltpu.sync_copy(dst_hbm.at[pl.ds(chunk_off(0), _C)], dstv0)
    pltpu.sync_copy(e_hbm.at[pl.ds(chunk_off(0), _C)], eb0)
    issue_gathers(0)
    pltpu.async_copy(e_hbm.at[pl.ds(0, _C)], vb1, semS1)

    def pair(u, _):
        steady(2 * u, 0)
        steady(2 * u + 1, 1)
        return 0

    lax.fori_loop(0, _NCHUNK // 2 - 1, pair, 0)
    steady(_NCHUNK - 2, 0)
    steady(_NCHUNK - 1, 1, last=True)
    wait_scatter(1)
    plsc.subcore_barrier()

    @pl.when(cid == 0)
    def _():
        pltpu.sync_copy(aggsh.at[pl.ds(zbase, _RPS)],
                        out0_hbm.at[pl.ds(zbase, _RPS)])

    @pl.when(cid == 1)
    def _():
        pltpu.sync_copy(aggsh.at[pl.ds(zbase, _RPS)],
                        out1_hbm.at[pl.ds(zbase, _RPS)])


def _sc_edge(k, q, v, e, src, dst):
    mesh = plsc.VectorSubcoreMesh(core_axis_name="c", subcore_axis_name="s")
    f = pl.kernel(
        _sc_edge_body,
        out_type=(jax.ShapeDtypeStruct((_NPAD, _D), jnp.float32),
                  jax.ShapeDtypeStruct((_NPAD, _D), jnp.float32)),
        mesh=mesh,
        scratch_types=(
            [pltpu.VMEM((_C,), jnp.int32)] * 4
            + [pltpu.VMEM((_C, _D), jnp.float32)] * 8
            + [pltpu.VMEM_SHARED((_NPAD, _D), jnp.float32)]
            + [pltpu.SemaphoreType.DMA] * 6
        ),
    )
    return f(k, q, v, e, src, dst)


def _epilogue_body(p0_ref, p1_ref, skip_ref, g_ref, b_ref, o_ref):
    out = p0_ref[...] + p1_ref[...] + skip_ref[...]
    mu = jnp.mean(out, axis=-1, keepdims=True)
    var = jnp.mean((out - mu) ** 2, axis=-1, keepdims=True)
    out = (out - mu) * jax.lax.rsqrt(var + 1e-5) * g_ref[...] + b_ref[...]
    o_ref[...] = out * 0.5 * (1.0 + jax.lax.erf(out / jnp.sqrt(2.0).astype(jnp.float32)))


def _epilogue(p0, p1, skip, ln_g_row, ln_b_row):
    return pl.pallas_call(
        _epilogue_body,
        grid=(_N // _NODE_BLK,),
        in_specs=[
            pl.BlockSpec((_NODE_BLK, _D), lambda i: (i, 0)),
            pl.BlockSpec((_NODE_BLK, _D), lambda i: (i, 0)),
            pl.BlockSpec((_NODE_BLK, _D), lambda i: (i, 0)),
            pl.BlockSpec((1, _D), lambda i: (0, 0)),
            pl.BlockSpec((1, _D), lambda i: (0, 0)),
        ],
        out_specs=pl.BlockSpec((_NODE_BLK, _D), lambda i: (i, 0)),
        out_shape=jax.ShapeDtypeStruct((_N, _D), jnp.float32),
    )(p0, p1, skip, ln_g_row, ln_b_row)


def kernel(x, edge_index, edge_attr, Wk, bk, Wq, bq, Wv, bv, We, be, Wskip,
           bias, ln_g, ln_b):
    src = edge_index[0].astype(jnp.int32)
    dst = edge_index[1].astype(jnp.int32)

    w_cat = jnp.concatenate([Wk.T, Wq.T, Wv.T, Wskip.T], axis=1)
    b_cat = jnp.concatenate([bk, bq, bv, bias])[None, :]
    kqvs = _node_matmuls(x, w_cat, b_cat)
    k = kqvs[:, :_D]
    q = kqvs[:, _D:2 * _D]
    v = kqvs[:, 2 * _D:3 * _D]
    skip = kqvs[:, 3 * _D:]

    e = _edge_matmul(edge_attr, We.T, be[None, :])

    p0, p1 = _sc_edge(k, q, v, e, src, dst)

    return _epilogue(p0, p1, skip, ln_g[None, :], ln_b[None, :])


# speedup vs baseline: 1.1120x; 1.1120x over previous
import functools

import numpy as np

import jax
import jax.numpy as jnp
from jax import lax
from jax.experimental import pallas as pl
from jax.experimental.pallas import tpu as pltpu
from jax.experimental.pallas import tpu_sc as plsc

_N = 10000
_E = 320000
_D = 128
_DE = 16
_DW = _D // 2

_NODE_BLK = 1000
_EDGE_BLK = 4000

_NC = 2
_NS = 16
_NW = _NC * _NS
_EPW = _E // _NW
_C = 40
_NCHUNK = _EPW // _C
_NPAD = 10240
_RPS = _NPAD // _NS

_PERM = (np.arange(0, 128, 32)[:, None]
         + np.concatenate([np.arange(0, 32, 2), np.arange(1, 32, 2)])[None, :]
         ).reshape(-1)
_INV = np.argsort(_PERM)


def _pack_cols(lo, hi):
    ue = lax.bitcast_convert_type(lo.astype(jnp.bfloat16), jnp.uint16)
    uo = lax.bitcast_convert_type(hi.astype(jnp.bfloat16), jnp.uint16)
    word = ue.astype(jnp.uint32) | (uo.astype(jnp.uint32) << 16)
    return lax.bitcast_convert_type(word, jnp.int32)


def _node_mm_body(x_ref, w_ref, b_ref, k_ref, qv_ref, s_ref):
    acc = jnp.dot(x_ref[...], w_ref[...], preferred_element_type=jnp.float32)
    acc = acc + b_ref[...]
    k_ref[...] = acc[:, 0:128]
    qv_ref[:, 0:64] = _pack_cols(acc[:, 128:192], acc[:, 192:256])
    qv_ref[:, 64:128] = _pack_cols(acc[:, 256:320], acc[:, 320:384])
    s_ref[...] = acc[:, 384:512]


def _node_matmuls(x, w_cat, b_cat):
    nspec = pl.BlockSpec((_NODE_BLK, _D), lambda i: (i, 0))
    return pl.pallas_call(
        _node_mm_body,
        grid=(_N // _NODE_BLK,),
        in_specs=[
            pl.BlockSpec((_NODE_BLK, _D), lambda i: (i, 0)),
            pl.BlockSpec((_D, 4 * _D), lambda i: (0, 0)),
            pl.BlockSpec((1, 4 * _D), lambda i: (0, 0)),
        ],
        out_specs=[nspec, nspec, nspec],
        out_shape=[jax.ShapeDtypeStruct((_N, _D), jnp.float32),
                   jax.ShapeDtypeStruct((_N, _D), jnp.int32),
                   jax.ShapeDtypeStruct((_N, _D), jnp.float32)],
    )(x, w_cat, b_cat)


def _edge_mm_body(a_ref, w_ref, b_ref, o_ref):
    acc = jnp.dot(a_ref[...], w_ref[...], preferred_element_type=jnp.float32)
    acc = acc + b_ref[...]
    o_ref[...] = _pack_cols(acc[:, 0:64], acc[:, 64:128])


def _edge_matmul(edge_attr, we_cat, be_cat):
    return pl.pallas_call(
        _edge_mm_body,
        grid=(_E // _EDGE_BLK,),
        in_specs=[
            pl.BlockSpec((_EDGE_BLK, _DE), lambda i: (i, 0)),
            pl.BlockSpec((_DE, _D), lambda i: (0, 0)),
            pl.BlockSpec((1, _D), lambda i: (0, 0)),
        ],
        out_specs=pl.BlockSpec((_EDGE_BLK, _DW), lambda i: (i, 0)),
        out_shape=jax.ShapeDtypeStruct((_E, _DW), jnp.int32),
    )(edge_attr, we_cat, be_cat)


def _sc_edge_body(k_hbm, qv_hbm, e_hbm, src_hbm, dst_hbm,
                  out0_hbm, out1_hbm,
                  srcv0, srcv1, dstv0, dstv1,
                  kb0, kb1, qvb0, qvb1, eb0, eb1, mb0, mb1,
                  aggsh, semG0, semG1, semE0, semE1, semS0, semS1):
    cid = lax.axis_index("c")
    sid = lax.axis_index("s")
    wid = sid * _NC + cid
    ebase = wid * _EPW

    srcv = (srcv0, srcv1)
    dstv = (dstv0, dstv1)
    kb = (kb0, kb1)
    qvb = (qvb0, qvb1)
    eb = (eb0, eb1)
    mb = (mb0, mb1)
    semG = (semG0, semG1)
    semE = (semE0, semE1)
    semS = (semS0, semS1)

    zero = jnp.zeros((16,), jnp.float32)

    def zrow(i, _):
        for r in range(_D // 16):
            mb0[i, pl.ds(r * 16, 16)] = zero
        return 0

    lax.fori_loop(0, _C, zrow, 0)
    zbase = pl.multiple_of(sid * _RPS, 8)
    for m in range(_RPS // _C):
        pltpu.sync_copy(mb0, aggsh.at[pl.ds(zbase + m * _C, _C)])
    plsc.subcore_barrier()

    def chunk_off(t):
        return pl.multiple_of(ebase + t * _C, 8)

    def issue_pre(t, b):
        off = chunk_off(t)
        pltpu.async_copy(e_hbm.at[pl.ds(off, _C)], eb[b], semE[b])
        pltpu.async_copy(src_hbm.at[pl.ds(off, _C)], srcv[b], semE[b])
        pltpu.async_copy(dst_hbm.at[pl.ds(off, _C)], dstv[b], semE[b])

    def wait_pre(b):
        pltpu.make_async_copy(e_hbm.at[pl.ds(0, _C)], eb[b], semE[b]).wait()
        pltpu.make_async_copy(src_hbm.at[pl.ds(0, _C)], srcv[b], semE[b]).wait()
        pltpu.make_async_copy(dst_hbm.at[pl.ds(0, _C)], dstv[b], semE[b]).wait()

    def issue_gathers(b):
        pltpu.async_copy(k_hbm.at[dstv[b]], kb[b], semG[b])
        pltpu.async_copy(qv_hbm.at[srcv[b]], qvb[b], semG[b])

    def wait_gathers(b):
        pltpu.make_async_copy(k_hbm.at[pl.ds(0, _C)], kb[b], semG[b]).wait()
        pltpu.make_async_copy(qv_hbm.at[pl.ds(0, _C)], qvb[b], semG[b]).wait()

    def compute(b):
        kp, qvp, ep, mp = kb[b], qvb[b], eb[b], mb[b]

        mask = jnp.int32(-65536)

        def unpk(w):
            lo = lax.bitcast_convert_type(w << 16, jnp.float32)
            hi = lax.bitcast_convert_type(w & mask, jnp.float32)
            return lo, hi

        def row(i, _):
            for g in range(_DW // 16):
                slw = pl.ds(g * 16, 16)
                ea, eo = unpk(ep[i, slw])
                qa, qo = unpk(qvp[i, slw])
                va, vo = unpk(qvp[i, pl.ds(64 + g * 16, 16)])
                za = kp[i, pl.ds(g * 32, 16)] + ea + qa
                zo = kp[i, pl.ds(g * 32 + 16, 16)] + eo + qo
                mp[i, pl.ds(g * 32, 16)] = va / (1.0 + jnp.exp(-za))
                mp[i, pl.ds(g * 32 + 16, 16)] = vo / (1.0 + jnp.exp(-zo))
            return 0

        lax.fori_loop(0, _C, row, 0)

    def issue_scatter(b):
        pltpu.async_copy(mb[b], aggsh.at[dstv[b]], semS[b], add=True)

    def wait_scatter(b):
        pltpu.make_async_copy(out0_hbm.at[pl.ds(0, _C)], mb[b], semS[b]).wait()

    def steady(t, p, last=False):
        q = 1 - p
        wait_scatter(q)
        if not last:
            issue_pre(t + 1, q)
        wait_gathers(p)
        compute(p)
        if not last:
            wait_pre(q)
            issue_gathers(q)
        issue_scatter(p)

    pltpu.sync_copy(src_hbm.at[pl.ds(chunk_off(0), _C)], srcv0)
    pltpu.sync_copy(dst_hbm.at[pl.ds(chunk_off(0), _C)], dstv0)
    pltpu.sync_copy(e_hbm.at[pl.ds(chunk_off(0), _C)], eb0)
    issue_gathers(0)
    pltpu.async_copy(out0_hbm.at[pl.ds(0, _C)], mb1, semS1)

    def pair(u, _):
        steady(2 * u, 0)
        steady(2 * u + 1, 1)
        return 0

    lax.fori_loop(0, _NCHUNK // 2 - 1, pair, 0)
    steady(_NCHUNK - 2, 0)
    steady(_NCHUNK - 1, 1, last=True)
    wait_scatter(1)
    plsc.subcore_barrier()

    @pl.when(cid == 0)
    def _():
        pltpu.sync_copy(aggsh.at[pl.ds(zbase, _RPS)],
                        out0_hbm.at[pl.ds(zbase, _RPS)])

    @pl.when(cid == 1)
    def _():
        pltpu.sync_copy(aggsh.at[pl.ds(zbase, _RPS)],
                        out1_hbm.at[pl.ds(zbase, _RPS)])


def _sc_edge(k, qv, e, src, dst):
    mesh = plsc.VectorSubcoreMesh(core_axis_name="c", subcore_axis_name="s")
    f = pl.kernel(
        _sc_edge_body,
        out_type=(jax.ShapeDtypeStruct((_NPAD, _D), jnp.float32),
                  jax.ShapeDtypeStruct((_NPAD, _D), jnp.float32)),
        mesh=mesh,
        scratch_types=(
            [pltpu.VMEM((_C,), jnp.int32)] * 4
            + [pltpu.VMEM((_C, _D), jnp.float32)] * 2
            + [pltpu.VMEM((_C, _D), jnp.int32)] * 2
            + [pltpu.VMEM((_C, _DW), jnp.int32)] * 2
            + [pltpu.VMEM((_C, _D), jnp.float32)] * 2
            + [pltpu.VMEM_SHARED((_NPAD, _D), jnp.float32)]
            + [pltpu.SemaphoreType.DMA] * 6
        ),
    )
    return f(k, qv, e, src, dst)


def _epilogue_body(p0_ref, p1_ref, skip_ref, g_ref, b_ref, o_ref):
    out = p0_ref[...] + p1_ref[...] + skip_ref[...]
    mu = jnp.mean(out, axis=-1, keepdims=True)
    var = jnp.mean((out - mu) ** 2, axis=-1, keepdims=True)
    out = (out - mu) * jax.lax.rsqrt(var + 1e-5) * g_ref[...] + b_ref[...]
    o_ref[...] = out * 0.5 * (1.0 + jax.lax.erf(out / jnp.sqrt(2.0).astype(jnp.float32)))


def _epilogue(p0, p1, skip, ln_g_row, ln_b_row):
    return pl.pallas_call(
        _epilogue_body,
        grid=(_N // _NODE_BLK,),
        in_specs=[
            pl.BlockSpec((_NODE_BLK, _D), lambda i: (i, 0)),
            pl.BlockSpec((_NODE_BLK, _D), lambda i: (i, 0)),
            pl.BlockSpec((_NODE_BLK, _D), lambda i: (i, 0)),
            pl.BlockSpec((1, _D), lambda i: (0, 0)),
            pl.BlockSpec((1, _D), lambda i: (0, 0)),
        ],
        out_specs=pl.BlockSpec((_NODE_BLK, _D), lambda i: (i, 0)),
        out_shape=jax.ShapeDtypeStruct((_N, _D), jnp.float32),
    )(p0, p1, skip, ln_g_row, ln_b_row)


def kernel(x, edge_index, edge_attr, Wk, bk, Wq, bq, Wv, bv, We, be, Wskip,
           bias, ln_g, ln_b):
    src = edge_index[0].astype(jnp.int32)
    dst = edge_index[1].astype(jnp.int32)

    w_cat = jnp.concatenate(
        [Wk[_PERM].T, Wq[0::2].T, Wq[1::2].T,
         Wv[0::2].T, Wv[1::2].T, Wskip[_PERM].T], axis=1)
    b_cat = jnp.concatenate(
        [bk[_PERM], bq[0::2], bq[1::2],
         bv[0::2], bv[1::2], bias[_PERM]])[None, :]
    k, qv, skip = _node_matmuls(x, w_cat, b_cat)

    we_cat = jnp.concatenate([We[0::2], We[1::2]], axis=0).T
    be_cat = jnp.concatenate([be[0::2], be[1::2]])[None, :]
    e = _edge_matmul(edge_attr, we_cat, be_cat)

    p0, p1 = _sc_edge(k, qv, e, src, dst)

    out = _epilogue(p0, p1, skip, ln_g[_PERM][None, :], ln_b[_PERM][None, :])
    return out[:, _INV]


# speedup vs baseline: 1.4957x; 1.3450x over previous
import functools

import numpy as np

import jax
import jax.numpy as jnp
from jax import lax
from jax.experimental import pallas as pl
from jax.experimental.pallas import tpu as pltpu
from jax.experimental.pallas import tpu_sc as plsc

_N = 10000
_E = 320000
_D = 128
_DE = 16
_DW = _D // 2

_NODE_BLK = 1000
_EDGE_BLK = 4000

_NC = 2
_NS = 16
_NW = _NC * _NS
_EPW = _E // _NW
_C = 40
_NCHUNK = _EPW // _C
_NPAD = 10240
_RPS = _NPAD // _NS

_PERM = (np.arange(0, 128, 32)[:, None]
         + np.concatenate([np.arange(0, 32, 2), np.arange(1, 32, 2)])[None, :]
         ).reshape(-1)
_INV = np.argsort(_PERM)


def _pack_cols(lo, hi):
    ue = lax.bitcast_convert_type(lo.astype(jnp.bfloat16), jnp.uint16)
    uo = lax.bitcast_convert_type(hi.astype(jnp.bfloat16), jnp.uint16)
    word = ue.astype(jnp.uint32) | (uo.astype(jnp.uint32) << 16)
    return lax.bitcast_convert_type(word, jnp.int32)


def _node_mm_body(x_ref, w_ref, b_ref, k_ref, qv_ref, s_ref):
    acc = jnp.dot(x_ref[...], w_ref[...], preferred_element_type=jnp.float32)
    acc = acc + b_ref[...]
    k_ref[...] = acc[:, 0:128]
    qv_ref[:, 0:64] = _pack_cols(acc[:, 128:192], acc[:, 192:256])
    qv_ref[:, 64:128] = _pack_cols(acc[:, 256:320], acc[:, 320:384])
    s_ref[...] = acc[:, 384:512]


def _node_matmuls(x, w_cat, b_cat):
    nspec = pl.BlockSpec((_NODE_BLK, _D), lambda i: (i, 0))
    return pl.pallas_call(
        _node_mm_body,
        grid=(_N // _NODE_BLK,),
        in_specs=[
            pl.BlockSpec((_NODE_BLK, _D), lambda i: (i, 0)),
            pl.BlockSpec((_D, 4 * _D), lambda i: (0, 0)),
            pl.BlockSpec((1, 4 * _D), lambda i: (0, 0)),
        ],
        out_specs=[nspec, nspec, nspec],
        out_shape=[jax.ShapeDtypeStruct((_N, _D), jnp.float32),
                   jax.ShapeDtypeStruct((_N, _D), jnp.int32),
                   jax.ShapeDtypeStruct((_N, _D), jnp.float32)],
    )(x, w_cat, b_cat)


def _edge_mm_body(a_ref, w_ref, b_ref, o_ref):
    acc = jnp.dot(a_ref[...], w_ref[...], preferred_element_type=jnp.float32)
    acc = acc + b_ref[...]
    o_ref[...] = _pack_cols(acc[:, 0:64], acc[:, 64:128])


def _edge_matmul(edge_attr, we_cat, be_cat):
    return pl.pallas_call(
        _edge_mm_body,
        grid=(_E // _EDGE_BLK,),
        in_specs=[
            pl.BlockSpec((_EDGE_BLK, _DE), lambda i: (i, 0)),
            pl.BlockSpec((_DE, _D), lambda i: (0, 0)),
            pl.BlockSpec((1, _D), lambda i: (0, 0)),
        ],
        out_specs=pl.BlockSpec((_EDGE_BLK, _DW), lambda i: (i, 0)),
        out_shape=jax.ShapeDtypeStruct((_E, _DW), jnp.int32),
    )(edge_attr, we_cat, be_cat)


def _sc_edge_body(k_hbm, qv_hbm, e_hbm, src_hbm, dst_hbm,
                  out0_hbm, out1_hbm,
                  srcv0, srcv1, srcv2, srcv3, dstv0, dstv1, dstv2, dstv3,
                  kb0, kb1, qvb0, qvb1, eb0, eb1, eb2, eb3, mb0,
                  aggsh, semG0, semG1,
                  semE0, semE1, semE2, semE3, semS0):
    cid = lax.axis_index("c")
    sid = lax.axis_index("s")
    wid = sid * _NC + cid
    ebase = wid * _EPW

    srcv = (srcv0, srcv1, srcv2, srcv3)
    dstv = (dstv0, dstv1, dstv2, dstv3)
    kb = (kb0, kb1)
    qvb = (qvb0, qvb1)
    eb = (eb0, eb1, eb2, eb3)
    semG = (semG0, semG1)
    semE = (semE0, semE1, semE2, semE3)

    zero = jnp.zeros((16,), jnp.float32)

    def zrow(i, _):
        for r in range(_D // 16):
            mb0[i, pl.ds(r * 16, 16)] = zero
        return 0

    lax.fori_loop(0, _C, zrow, 0)
    zbase = pl.multiple_of(sid * _RPS, 8)
    for m in range(_RPS // _C):
        pltpu.sync_copy(mb0, aggsh.at[pl.ds(zbase + m * _C, _C)])
    plsc.subcore_barrier()

    def chunk_off(t):
        return pl.multiple_of(ebase + t * _C, 8)

    def issue_pre(t, s):
        off = chunk_off(t)
        pltpu.async_copy(e_hbm.at[pl.ds(off, _C)], eb[s], semE[s])
        pltpu.async_copy(src_hbm.at[pl.ds(off, _C)], srcv[s], semE[s])
        pltpu.async_copy(dst_hbm.at[pl.ds(off, _C)], dstv[s], semE[s])

    def wait_pre(s):
        pltpu.make_async_copy(e_hbm.at[pl.ds(0, _C)], eb[s], semE[s]).wait()
        pltpu.make_async_copy(src_hbm.at[pl.ds(0, _C)], srcv[s], semE[s]).wait()
        pltpu.make_async_copy(dst_hbm.at[pl.ds(0, _C)], dstv[s], semE[s]).wait()

    def issue_gathers(b, s):
        pltpu.async_copy(k_hbm.at[dstv[s]], kb[b], semG[b])
        pltpu.async_copy(qv_hbm.at[srcv[s]], qvb[b], semG[b])

    def wait_gathers(b):
        pltpu.make_async_copy(k_hbm.at[pl.ds(0, _C)], kb[b], semG[b]).wait()
        pltpu.make_async_copy(qv_hbm.at[pl.ds(0, _C)], qvb[b], semG[b]).wait()

    def compute(b, s):
        kp, qvp, ep, mp = kb[b], qvb[b], eb[s], mb0

        mask = jnp.int32(-65536)

        def unpk(w):
            lo = lax.bitcast_convert_type(w << 16, jnp.float32)
            hi = lax.bitcast_convert_type(w & mask, jnp.float32)
            return lo, hi

        def row(i, _):
            for g in range(_DW // 16):
                slw = pl.ds(g * 16, 16)
                ea, eo = unpk(ep[i, slw])
                qa, qo = unpk(qvp[i, slw])
                va, vo = unpk(qvp[i, pl.ds(64 + g * 16, 16)])
                za = kp[i, pl.ds(g * 32, 16)] + ea + qa
                zo = kp[i, pl.ds(g * 32 + 16, 16)] + eo + qo
                mp[i, pl.ds(g * 32, 16)] = va / (1.0 + jnp.exp(-za))
                mp[i, pl.ds(g * 32 + 16, 16)] = vo / (1.0 + jnp.exp(-zo))
            return 0

        lax.fori_loop(0, _C, row, 0)

    def issue_scatter(s):
        pltpu.async_copy(mb0, aggsh.at[dstv[s]], semS0, add=True)

    def wait_scatter():
        pltpu.make_async_copy(out0_hbm.at[pl.ds(0, _C)], mb0, semS0).wait()

    def steady(t, j, pre2=True, gat1=True):
        p = j & 1
        q = 1 - p
        if gat1:
            wait_pre((j + 1) % 4)
            issue_gathers(q, (j + 1) % 4)
        if pre2:
            issue_pre(t + 2, (j + 2) % 4)
        wait_gathers(p)
        wait_scatter()
        compute(p, j)
        issue_scatter(j)

    pltpu.sync_copy(src_hbm.at[pl.ds(chunk_off(0), _C)], srcv0)
    pltpu.sync_copy(dst_hbm.at[pl.ds(chunk_off(0), _C)], dstv0)
    pltpu.sync_copy(e_hbm.at[pl.ds(chunk_off(0), _C)], eb0)
    issue_gathers(0, 0)
    issue_pre(1, 1)
    pltpu.async_copy(out0_hbm.at[pl.ds(0, _C)], mb0, semS0)

    def quad(u, _):
        t0 = 4 * u
        steady(t0, 0)
        steady(t0 + 1, 1)
        steady(t0 + 2, 2)
        steady(t0 + 3, 3)
        return 0

    lax.fori_loop(0, (_NCHUNK - 2) // 4, quad, 0)
    steady(_NCHUNK - 2, 0, pre2=False)
    steady(_NCHUNK - 1, 1, pre2=False, gat1=False)
    wait_scatter()
    plsc.subcore_barrier()

    @pl.when(cid == 0)
    def _():
        pltpu.sync_copy(aggsh.at[pl.ds(zbase, _RPS)],
                        out0_hbm.at[pl.ds(zbase, _RPS)])

    @pl.when(cid == 1)
    def _():
        pltpu.sync_copy(aggsh.at[pl.ds(zbase, _RPS)],
                        out1_hbm.at[pl.ds(zbase, _RPS)])


def _sc_edge(k, qv, e, src, dst):
    mesh = plsc.VectorSubcoreMesh(core_axis_name="c", subcore_axis_name="s")
    f = pl.kernel(
        _sc_edge_body,
        out_type=(jax.ShapeDtypeStruct((_NPAD, _D), jnp.float32),
                  jax.ShapeDtypeStruct((_NPAD, _D), jnp.float32)),
        mesh=mesh,
        scratch_types=(
            [pltpu.VMEM((_C,), jnp.int32)] * 8
            + [pltpu.VMEM((_C, _D), jnp.float32)] * 2
            + [pltpu.VMEM((_C, _D), jnp.int32)] * 2
            + [pltpu.VMEM((_C, _DW), jnp.int32)] * 4
            + [pltpu.VMEM((_C, _D), jnp.float32)] * 1
            + [pltpu.VMEM_SHARED((_NPAD, _D), jnp.float32)]
            + [pltpu.SemaphoreType.DMA] * 7
        ),
    )
    return f(k, qv, e, src, dst)


def _epilogue_body(p0_ref, p1_ref, skip_ref, g_ref, b_ref, o_ref):
    out = p0_ref[...] + p1_ref[...] + skip_ref[...]
    mu = jnp.mean(out, axis=-1, keepdims=True)
    var = jnp.mean((out - mu) ** 2, axis=-1, keepdims=True)
    out = (out - mu) * jax.lax.rsqrt(var + 1e-5) * g_ref[...] + b_ref[...]
    o_ref[...] = out * 0.5 * (1.0 + jax.lax.erf(out / jnp.sqrt(2.0).astype(jnp.float32)))


def _epilogue(p0, p1, skip, ln_g_row, ln_b_row):
    return pl.pallas_call(
        _epilogue_body,
        grid=(_N // _NODE_BLK,),
        in_specs=[
            pl.BlockSpec((_NODE_BLK, _D), lambda i: (i, 0)),
            pl.BlockSpec((_NODE_BLK, _D), lambda i: (i, 0)),
            pl.BlockSpec((_NODE_BLK, _D), lambda i: (i, 0)),
            pl.BlockSpec((1, _D), lambda i: (0, 0)),
            pl.BlockSpec((1, _D), lambda i: (0, 0)),
        ],
        out_specs=pl.BlockSpec((_NODE_BLK, _D), lambda i: (i, 0)),
        out_shape=jax.ShapeDtypeStruct((_N, _D), jnp.float32),
    )(p0, p1, skip, ln_g_row, ln_b_row)


def kernel(x, edge_index, edge_attr, Wk, bk, Wq, bq, Wv, bv, We, be, Wskip,
           bias, ln_g, ln_b):
    src = edge_index[0].astype(jnp.int32)
    dst = edge_index[1].astype(jnp.int32)

    w_cat = jnp.concatenate(
        [Wk[_PERM].T, Wq[0::2].T, Wq[1::2].T,
         Wv[0::2].T, Wv[1::2].T, Wskip[_PERM].T], axis=1)
    b_cat = jnp.concatenate(
        [bk[_PERM], bq[0::2], bq[1::2],
         bv[0::2], bv[1::2], bias[_PERM]])[None, :]
    k, qv, skip = _node_matmuls(x, w_cat, b_cat)

    we_cat = jnp.concatenate([We[0::2], We[1::2]], axis=0).T
    be_cat = jnp.concatenate([be[0::2], be[1::2]])[None, :]
    e = _edge_matmul(edge_attr, we_cat, be_cat)

    p0, p1 = _sc_edge(k, qv, e, src, dst)

    out = _epilogue(p0, p1, skip, ln_g[_PERM][None, :], ln_b[_PERM][None, :])
    return out[:, _INV]


# speedup vs baseline: 1.4983x; 1.0018x over previous
import functools

import numpy as np

import jax
import jax.numpy as jnp
from jax import lax
from jax.experimental import pallas as pl
from jax.experimental.pallas import tpu as pltpu
from jax.experimental.pallas import tpu_sc as plsc

_N = 10000
_E = 320000
_D = 128
_DE = 16
_DW = _D // 2

_NODE_BLK = 1000
_EDGE_BLK = 4000

_NC = 2
_NS = 16
_NW = _NC * _NS
_EPW = _E // _NW
_C = 40
_NCHUNK = _EPW // _C
_NPAD = 10240
_RPS = _NPAD // _NS

_PERM = (np.arange(0, 128, 32)[:, None]
         + np.concatenate([np.arange(0, 32, 2), np.arange(1, 32, 2)])[None, :]
         ).reshape(-1)
_INV = np.argsort(_PERM)


def _pack_cols(lo, hi):
    ue = lax.bitcast_convert_type(lo.astype(jnp.bfloat16), jnp.uint16)
    uo = lax.bitcast_convert_type(hi.astype(jnp.bfloat16), jnp.uint16)
    word = ue.astype(jnp.uint32) | (uo.astype(jnp.uint32) << 16)
    return lax.bitcast_convert_type(word, jnp.int32)


def _node_mm_body(x_ref, w_ref, b_ref, k_ref, qv_ref, s_ref):
    acc = jnp.dot(x_ref[...], w_ref[...], preferred_element_type=jnp.float32)
    acc = acc + b_ref[...]
    k_ref[...] = acc[:, 0:128]
    qv_ref[:, 0:64] = _pack_cols(acc[:, 128:192], acc[:, 192:256])
    qv_ref[:, 64:128] = _pack_cols(acc[:, 256:320], acc[:, 320:384])
    s_ref[...] = acc[:, 384:512]


def _node_matmuls(x, w_cat, b_cat):
    nspec = pl.BlockSpec((_NODE_BLK, _D), lambda i: (i, 0))
    return pl.pallas_call(
        _node_mm_body,
        grid=(_N // _NODE_BLK,),
        in_specs=[
            pl.BlockSpec((_NODE_BLK, _D), lambda i: (i, 0)),
            pl.BlockSpec((_D, 4 * _D), lambda i: (0, 0)),
            pl.BlockSpec((1, 4 * _D), lambda i: (0, 0)),
        ],
        out_specs=[nspec, nspec, nspec],
        out_shape=[jax.ShapeDtypeStruct((_N, _D), jnp.float32),
                   jax.ShapeDtypeStruct((_N, _D), jnp.int32),
                   jax.ShapeDtypeStruct((_N, _D), jnp.float32)],
    )(x, w_cat, b_cat)


def _edge_mm_body(a_ref, w_ref, b_ref, o_ref):
    acc = jnp.dot(a_ref[...], w_ref[...], preferred_element_type=jnp.float32)
    acc = acc + b_ref[...]
    o_ref[...] = _pack_cols(acc[:, 0:64], acc[:, 64:128])


def _edge_matmul(edge_attr, we_cat, be_cat):
    return pl.pallas_call(
        _edge_mm_body,
        grid=(_E // _EDGE_BLK,),
        in_specs=[
            pl.BlockSpec((_EDGE_BLK, _DE), lambda i: (i, 0)),
            pl.BlockSpec((_DE, _D), lambda i: (0, 0)),
            pl.BlockSpec((1, _D), lambda i: (0, 0)),
        ],
        out_specs=pl.BlockSpec((_EDGE_BLK, _DW), lambda i: (i, 0)),
        out_shape=jax.ShapeDtypeStruct((_E, _DW), jnp.int32),
    )(edge_attr, we_cat, be_cat)


def _sc_edge_body(k_hbm, qv_hbm, e_hbm, src_hbm, dst_hbm,
                  out0_hbm, out1_hbm,
                  srcv0, srcv1, srcv2, srcv3, dstv0, dstv1, dstv2, dstv3,
                  kb0, kb1, qvb0, qvb1, eb0, eb1, eb2, eb3, mb0,
                  aggsh, semG0, semG1,
                  semE0, semE1, semE2, semE3, semS0):
    cid = lax.axis_index("c")
    sid = lax.axis_index("s")
    wid = sid * _NC + cid
    ebase = wid * _EPW

    srcv = (srcv0, srcv1, srcv2, srcv3)
    dstv = (dstv0, dstv1, dstv2, dstv3)
    kb = (kb0, kb1)
    qvb = (qvb0, qvb1)
    eb = (eb0, eb1, eb2, eb3)
    semG = (semG0, semG1)
    semE = (semE0, semE1, semE2, semE3)

    zero = jnp.zeros((16,), jnp.float32)

    def zrow(i, _):
        for r in range(_D // 16):
            mb0[i, pl.ds(r * 16, 16)] = zero
        return 0

    lax.fori_loop(0, _C, zrow, 0)
    zbase = pl.multiple_of(sid * _RPS, 8)
    for m in range(_RPS // _C):
        pltpu.sync_copy(mb0, aggsh.at[pl.ds(zbase + m * _C, _C)])
    plsc.subcore_barrier()

    def chunk_off(t):
        return pl.multiple_of(ebase + t * _C, 8)

    def issue_pre(t, s):
        off = chunk_off(t)
        pltpu.async_copy(e_hbm.at[pl.ds(off, _C)], eb[s], semE[s])
        pltpu.async_copy(src_hbm.at[pl.ds(off, _C)], srcv[s], semE[s])
        pltpu.async_copy(dst_hbm.at[pl.ds(off, _C)], dstv[s], semE[s])

    def wait_pre(s):
        pltpu.make_async_copy(e_hbm.at[pl.ds(0, _C)], eb[s], semE[s]).wait()
        pltpu.make_async_copy(src_hbm.at[pl.ds(0, _C)], srcv[s], semE[s]).wait()
        pltpu.make_async_copy(dst_hbm.at[pl.ds(0, _C)], dstv[s], semE[s]).wait()

    def issue_gathers(b, s):
        pltpu.async_copy(k_hbm.at[dstv[s]], kb[b], semG[b])
        pltpu.async_copy(qv_hbm.at[srcv[s]], qvb[b], semG[b])

    def wait_gathers(b):
        pltpu.make_async_copy(k_hbm.at[pl.ds(0, _C)], kb[b], semG[b]).wait()
        pltpu.make_async_copy(qv_hbm.at[pl.ds(0, _C)], qvb[b], semG[b]).wait()

    def compute(b, s):
        kp, qvp, ep, mp = kb[b], qvb[b], eb[s], mb0

        mask = jnp.int32(-65536)

        def unpk(w):
            lo = lax.bitcast_convert_type(w << 16, jnp.float32)
            hi = lax.bitcast_convert_type(w & mask, jnp.float32)
            return lo, hi

        @plsc.parallel_loop(0, _C, 1)
        def row(i):
            for g in range(_DW // 16):
                slw = pl.ds(g * 16, 16)
                ea, eo = unpk(ep[i, slw])
                qa, qo = unpk(qvp[i, slw])
                va, vo = unpk(qvp[i, pl.ds(64 + g * 16, 16)])
                za = kp[i, pl.ds(g * 32, 16)] + ea + qa
                zo = kp[i, pl.ds(g * 32 + 16, 16)] + eo + qo
                mp[i, pl.ds(g * 32, 16)] = va / (1.0 + jnp.exp(-za))
                mp[i, pl.ds(g * 32 + 16, 16)] = vo / (1.0 + jnp.exp(-zo))

    def issue_scatter(s):
        pltpu.async_copy(mb0, aggsh.at[dstv[s]], semS0, add=True)

    def wait_scatter():
        pltpu.make_async_copy(out0_hbm.at[pl.ds(0, _C)], mb0, semS0).wait()

    def steady(t, j, pre2=True, gat1=True):
        p = j & 1
        q = 1 - p
        if gat1:
            wait_pre((j + 1) % 4)
            issue_gathers(q, (j + 1) % 4)
        if pre2:
            issue_pre(t + 2, (j + 2) % 4)
        wait_gathers(p)
        wait_scatter()
        compute(p, j)
        issue_scatter(j)

    pltpu.sync_copy(src_hbm.at[pl.ds(chunk_off(0), _C)], srcv0)
    pltpu.sync_copy(dst_hbm.at[pl.ds(chunk_off(0), _C)], dstv0)
    pltpu.sync_copy(e_hbm.at[pl.ds(chunk_off(0), _C)], eb0)
    issue_gathers(0, 0)
    issue_pre(1, 1)
    pltpu.async_copy(out0_hbm.at[pl.ds(0, _C)], mb0, semS0)

    def quad(u, _):
        t0 = 4 * u
        steady(t0, 0)
        steady(t0 + 1, 1)
        steady(t0 + 2, 2)
        steady(t0 + 3, 3)
        return 0

    lax.fori_loop(0, (_NCHUNK - 2) // 4, quad, 0)
    steady(_NCHUNK - 2, 0, pre2=False)
    steady(_NCHUNK - 1, 1, pre2=False, gat1=False)
    wait_scatter()
    plsc.subcore_barrier()

    @pl.when(cid == 0)
    def _():
        pltpu.sync_copy(aggsh.at[pl.ds(zbase, _RPS)],
                        out0_hbm.at[pl.ds(zbase, _RPS)])

    @pl.when(cid == 1)
    def _():
        pltpu.sync_copy(aggsh.at[pl.ds(zbase, _RPS)],
                        out1_hbm.at[pl.ds(zbase, _RPS)])


def _sc_edge(k, qv, e, src, dst):
    mesh = plsc.VectorSubcoreMesh(core_axis_name="c", subcore_axis_name="s")
    f = pl.kernel(
        _sc_edge_body,
        out_type=(jax.ShapeDtypeStruct((_NPAD, _D), jnp.float32),
                  jax.ShapeDtypeStruct((_NPAD, _D), jnp.float32)),
        mesh=mesh,
        scratch_types=(
            [pltpu.VMEM((_C,), jnp.int32)] * 8
            + [pltpu.VMEM((_C, _D), jnp.float32)] * 2
            + [pltpu.VMEM((_C, _D), jnp.int32)] * 2
            + [pltpu.VMEM((_C, _DW), jnp.int32)] * 4
            + [pltpu.VMEM((_C, _D), jnp.float32)] * 1
            + [pltpu.VMEM_SHARED((_NPAD, _D), jnp.float32)]
            + [pltpu.SemaphoreType.DMA] * 7
        ),
    )
    return f(k, qv, e, src, dst)


def _epilogue_body(p0_ref, p1_ref, skip_ref, g_ref, b_ref, o_ref):
    out = p0_ref[...] + p1_ref[...] + skip_ref[...]
    mu = jnp.mean(out, axis=-1, keepdims=True)
    var = jnp.mean((out - mu) ** 2, axis=-1, keepdims=True)
    out = (out - mu) * jax.lax.rsqrt(var + 1e-5) * g_ref[...] + b_ref[...]
    o_ref[...] = out * 0.5 * (1.0 + jax.lax.erf(out / jnp.sqrt(2.0).astype(jnp.float32)))


def _epilogue(p0, p1, skip, ln_g_row, ln_b_row):
    return pl.pallas_call(
        _epilogue_body,
        grid=(_N // _NODE_BLK,),
        in_specs=[
            pl.BlockSpec((_NODE_BLK, _D), lambda i: (i, 0)),
            pl.BlockSpec((_NODE_BLK, _D), lambda i: (i, 0)),
            pl.BlockSpec((_NODE_BLK, _D), lambda i: (i, 0)),
            pl.BlockSpec((1, _D), lambda i: (0, 0)),
            pl.BlockSpec((1, _D), lambda i: (0, 0)),
        ],
        out_specs=pl.BlockSpec((_NODE_BLK, _D), lambda i: (i, 0)),
        out_shape=jax.ShapeDtypeStruct((_N, _D), jnp.float32),
    )(p0, p1, skip, ln_g_row, ln_b_row)


def kernel(x, edge_index, edge_attr, Wk, bk, Wq, bq, Wv, bv, We, be, Wskip,
           bias, ln_g, ln_b):
    src = edge_index[0].astype(jnp.int32)
    dst = edge_index[1].astype(jnp.int32)

    w_cat = jnp.concatenate(
        [Wk[_PERM].T, Wq[0::2].T, Wq[1::2].T,
         Wv[0::2].T, Wv[1::2].T, Wskip[_PERM].T], axis=1)
    b_cat = jnp.concatenate(
        [bk[_PERM], bq[0::2], bq[1::2],
         bv[0::2], bv[1::2], bias[_PERM]])[None, :]
    k, qv, skip = _node_matmuls(x, w_cat, b_cat)

    we_cat = jnp.concatenate([We[0::2], We[1::2]], axis=0).T
    be_cat = jnp.concatenate([be[0::2], be[1::2]])[None, :]
    e = _edge_matmul(edge_attr, we_cat, be_cat)

    p0, p1 = _sc_edge(k, qv, e, src, dst)

    out = _epilogue(p0, p1, skip, ln_g[_PERM][None, :], ln_b[_PERM][None, :])
    return out[:, _INV]
